# baseline (speedup 1.0000x reference)
import jax
import jax.numpy as jnp
from jax.experimental import pallas as pl
from jax.experimental.pallas import tpu as pltpu

B = 4
N = 1024
NUM_HEADS = 4
F_HID = 128
H_DUGAT = 64
D = F_HID * 2
HF = NUM_HEADS * H_DUGAT
HB = 128

NB = 512
NI = N // NB


def _gat_kernel(x_ref, a_ref, mask_ref, e_ref, np_ref, w_ref, acat_ref,
                out_ref, haug_scr, es_scr, edt_scr, m_scr):
    i = pl.program_id(1)

    @pl.when(i == 0)
    def _per_batch_setup():
        recx = x_ref[0] * np_ref[...] * e_ref[0]
        h_all = jnp.dot(recx, w_ref[...], preferred_element_type=jnp.float32)
        for h in range(NUM_HEADS):
            haug_scr[:, h * HB:h * HB + H_DUGAT] = \
                h_all[:, h * H_DUGAT:(h + 1) * H_DUGAT]
            haug_scr[:, h * HB + H_DUGAT:(h + 1) * HB] = \
                jnp.ones((N, HB - H_DUGAT), jnp.float32)
        esd = jnp.dot(h_all, acat_ref[...], preferred_element_type=jnp.float32)
        es_scr[:, 0:8] = jnp.exp(esd)
        es_scr[:, 8:16] = jnp.exp(0.2 * esd)
        edt = esd.T
        edt_scr[0:8, :] = jnp.exp(edt)
        edt_scr[8:16, :] = jnp.exp(0.2 * edt)

    adjf = (a_ref[0] > 0.5).astype(jnp.float32)
    es_blk = es_scr[pl.ds(i * NB, NB), :]
    mask = mask_ref[0]
    for h in range(NUM_HEADS):
        e1 = es_blk[:, h:h + 1]
        e2 = es_blk[:, 8 + h:9 + h]
        f1 = edt_scr[NUM_HEADS + h, :][None, :]
        f2 = edt_scr[12 + h, :][None, :]
        q1 = jnp.dot(e1, f1, preferred_element_type=jnp.float32)
        q2 = jnp.dot(e2, f2, preferred_element_type=jnp.float32)
        p = jnp.maximum(q1, q2) * adjf
        oa = jnp.dot(p, haug_scr[:, h * HB:(h + 1) * HB],
                     preferred_element_type=jnp.float32)
        out_h = oa[:, :H_DUGAT] / oa[:, H_DUGAT:H_DUGAT + 1]
        out_h = jnp.where(out_h > 0, out_h, jnp.exp(out_h) - 1.0)
        out_ref[0, :, h * H_DUGAT:(h + 1) * H_DUGAT] = out_h * mask


@jax.jit
def kernel(x, A, mask_zero, p_atten, e_atten, Npost, W, a_src, a_dst):
    del p_atten
    x3 = x[:, None, :]
    w_flat = jnp.transpose(W, (1, 0, 2)).reshape(D, HF)
    acat = jnp.zeros((HF, 8), jnp.float32)
    for h in range(NUM_HEADS):
        acat = acat.at[h * H_DUGAT:(h + 1) * H_DUGAT, h].set(a_src[h])
        acat = acat.at[h * H_DUGAT:(h + 1) * H_DUGAT, NUM_HEADS + h].set(a_dst[h])

    grid = (B, NI)
    return pl.pallas_call(
        _gat_kernel,
        grid=grid,
        compiler_params=pltpu.CompilerParams(
            dimension_semantics=("parallel", "arbitrary")),
        in_specs=[
            pl.BlockSpec((1, 1, D), lambda b, i: (b, 0, 0)),
            pl.BlockSpec((1, NB, N), lambda b, i: (b, i, 0)),
            pl.BlockSpec((1, NB, 1), lambda b, i: (b, i, 0)),
            pl.BlockSpec((1, N, D), lambda b, i: (b, 0, 0)),
            pl.BlockSpec((N, D), lambda b, i: (0, 0)),
            pl.BlockSpec((D, HF), lambda b, i: (0, 0)),
            pl.BlockSpec((HF, 8), lambda b, i: (0, 0)),
        ],
        out_specs=pl.BlockSpec((1, NB, HF), lambda b, i: (b, i, 0)),
        out_shape=jax.ShapeDtypeStruct((B, N, HF), jnp.float32),
        scratch_shapes=[
            pltpu.VMEM((N, NUM_HEADS * HB), jnp.float32),
            pltpu.VMEM((N, 16), jnp.float32),
            pltpu.VMEM((16, N), jnp.float32),
            pltpu.VMEM((8, 1), jnp.float32),
        ],
    )(x3, A, mask_zero, e_atten, Npost, w_flat, acat)


# speedup vs baseline: 1.4056x; 1.4056x over previous
import jax
import jax.numpy as jnp
from jax.experimental import pallas as pl
from jax.experimental.pallas import tpu as pltpu

B = 4
N = 1024
NUM_HEADS = 4
F_HID = 128
H_DUGAT = 64
D = F_HID * 2
HF = NUM_HEADS * H_DUGAT
HB = 128

NB = 512
NI = N // NB


def _gat_kernel(x_ref, a_ref, mask_ref, e_ref, np_ref, w_ref, acat_ref,
                out_ref, haug_scr, es_scr, edt_scr, m_scr):
    i = pl.program_id(1)

    @pl.when(i == 0)
    def _per_batch_setup():
        recx = x_ref[0] * np_ref[...] * e_ref[0]
        h_all = jnp.dot(recx, w_ref[...], preferred_element_type=jnp.float32)
        for h in range(NUM_HEADS):
            haug_scr[:, h * HB:h * HB + H_DUGAT] = \
                h_all[:, h * H_DUGAT:(h + 1) * H_DUGAT]
            haug_scr[:, h * HB + H_DUGAT:(h + 1) * HB] = \
                jnp.ones((N, HB - H_DUGAT), jnp.float32)
        esd = jnp.dot(h_all, acat_ref[...], preferred_element_type=jnp.float32)
        es_scr[:, 0:8] = jnp.exp(0.8 * esd)
        edt = esd.T
        edt_scr[0:8, :] = jnp.exp(0.8 * edt)
        edt_scr[8:16, :] = jnp.exp(0.2 * edt)

    adjf = (a_ref[0] > 0.5).astype(jnp.float32)
    es_blk = es_scr[pl.ds(i * NB, NB), :]
    mask = mask_ref[0]
    for h in range(NUM_HEADS):
        rho = es_blk[:, h:h + 1]
        gam = edt_scr[NUM_HEADS + h, :][None, :]
        f2 = edt_scr[12 + h, :][None, :]
        p = jnp.maximum(rho * gam, 1.0) * (adjf * f2)
        oa = jnp.dot(p, haug_scr[:, h * HB:(h + 1) * HB],
                     preferred_element_type=jnp.float32)
        out_h = oa[:, :H_DUGAT] / oa[:, H_DUGAT:H_DUGAT + 1]
        out_h = jnp.where(out_h > 0, out_h, jnp.exp(out_h) - 1.0)
        out_ref[0, :, h * H_DUGAT:(h + 1) * H_DUGAT] = out_h * mask


@jax.jit
def kernel(x, A, mask_zero, p_atten, e_atten, Npost, W, a_src, a_dst):
    del p_atten
    x3 = x[:, None, :]
    w_flat = jnp.transpose(W, (1, 0, 2)).reshape(D, HF)
    acat = jnp.zeros((HF, 8), jnp.float32)
    for h in range(NUM_HEADS):
        acat = acat.at[h * H_DUGAT:(h + 1) * H_DUGAT, h].set(a_src[h])
        acat = acat.at[h * H_DUGAT:(h + 1) * H_DUGAT, NUM_HEADS + h].set(a_dst[h])

    grid = (B, NI)
    return pl.pallas_call(
        _gat_kernel,
        grid=grid,
        compiler_params=pltpu.CompilerParams(
            dimension_semantics=("parallel", "arbitrary")),
        in_specs=[
            pl.BlockSpec((1, 1, D), lambda b, i: (b, 0, 0)),
            pl.BlockSpec((1, NB, N), lambda b, i: (b, i, 0)),
            pl.BlockSpec((1, NB, 1), lambda b, i: (b, i, 0)),
            pl.BlockSpec((1, N, D), lambda b, i: (b, 0, 0)),
            pl.BlockSpec((N, D), lambda b, i: (0, 0)),
            pl.BlockSpec((D, HF), lambda b, i: (0, 0)),
            pl.BlockSpec((HF, 8), lambda b, i: (0, 0)),
        ],
        out_specs=pl.BlockSpec((1, NB, HF), lambda b, i: (b, i, 0)),
        out_shape=jax.ShapeDtypeStruct((B, N, HF), jnp.float32),
        scratch_shapes=[
            pltpu.VMEM((N, NUM_HEADS * HB), jnp.float32),
            pltpu.VMEM((N, 8), jnp.float32),
            pltpu.VMEM((16, N), jnp.float32),
            pltpu.VMEM((8, 1), jnp.float32),
        ],
    )(x3, A, mask_zero, e_atten, Npost, w_flat, acat)
